# baseline (speedup 1.0000x reference)
import functools

import jax
import jax.numpy as jnp
from jax import lax
from jax.experimental import pallas as pl
from jax.experimental.pallas import tpu as pltpu
from jax.experimental.pallas import tpu_sc as plsc

F32 = jnp.float32

_NC = 2
_NS = 16
_NW = _NC * _NS


def _tc1_body(x_ref, w_ref, asm_ref, adm_ref, xwa_ref, ad_ref):
    d = w_ref.shape[1]
    xw = jnp.dot(x_ref[...], w_ref[...], preferred_element_type=F32)
    xwa_ref[:, :d] = xw
    xwa_ref[:, d:] = jnp.dot(xw, asm_ref[...], preferred_element_type=F32)
    ad_ref[...] = jnp.dot(xw, adm_ref[...], preferred_element_type=F32)


def _run_tc1(x, W, Asm, Adm, blk):
    n, f_in = x.shape
    d = W.shape[1]
    grid = n // blk
    return pl.pallas_call(
        _tc1_body,
        grid=(grid,),
        in_specs=[
            pl.BlockSpec((blk, f_in), lambda i: (i, 0)),
            pl.BlockSpec((f_in, d), lambda i: (0, 0)),
            pl.BlockSpec((d, 16), lambda i: (0, 0)),
            pl.BlockSpec((d, 16), lambda i: (0, 0)),
        ],
        out_specs=[
            pl.BlockSpec((blk, d + 16), lambda i: (i, 0)),
            pl.BlockSpec((blk, 16), lambda i: (i, 0)),
        ],
        out_shape=[
            jax.ShapeDtypeStruct((n, d + 16), F32),
            jax.ShapeDtypeStruct((n, 16), F32),
        ],
    )(x, W, Asm, Adm)


def _tc_mid_body(comb_ref, xwa1_ref, ad1_ref, b1_ref, e16_ref, w2_ref,
                 asm2_ref, adm2_ref, xwa2_ref, ad2_ref):
    d1 = w2_ref.shape[0]
    d2 = w2_ref.shape[1]
    a = xwa1_ref[:, d1:] + ad1_ref[...]
    wself = jnp.exp(jnp.where(a > 0, a, 0.2 * a))
    den16 = comb_ref[0, :, d1:] + comb_ref[1, :, d1:] + wself
    wself_x = jnp.dot(wself, e16_ref[...], preferred_element_type=F32)
    num = (comb_ref[0, :, :d1] + comb_ref[1, :, :d1]
           + xwa1_ref[:, :d1] * wself_x)
    den = jnp.dot(den16, e16_ref[...], preferred_element_type=F32)
    h = num / den + b1_ref[...]
    h = jnp.where(h > 0, h, jnp.exp(h) - 1.0)
    xw2 = jnp.dot(h, w2_ref[...], preferred_element_type=F32)
    xwa2_ref[:, :d2] = xw2
    xwa2_ref[:, d2:] = jnp.dot(xw2, asm2_ref[...], preferred_element_type=F32)
    ad2_ref[...] = jnp.dot(xw2, adm2_ref[...], preferred_element_type=F32)


def _run_tc_mid(comb, xwa1, ad1, b1, E16, W2, Asm2, Adm2, blk):
    n = xwa1.shape[0]
    d1, d2 = W2.shape
    grid = n // blk
    return pl.pallas_call(
        _tc_mid_body,
        grid=(grid,),
        in_specs=[
            pl.BlockSpec((2, blk, d1 + 16), lambda i: (0, i, 0)),
            pl.BlockSpec((blk, d1 + 16), lambda i: (i, 0)),
            pl.BlockSpec((blk, 16), lambda i: (i, 0)),
            pl.BlockSpec((1, d1), lambda i: (0, 0)),
            pl.BlockSpec((16, d1), lambda i: (0, 0)),
            pl.BlockSpec((d1, d2), lambda i: (0, 0)),
            pl.BlockSpec((d2, 16), lambda i: (0, 0)),
            pl.BlockSpec((d2, 16), lambda i: (0, 0)),
        ],
        out_specs=[
            pl.BlockSpec((blk, d2 + 16), lambda i: (i, 0)),
            pl.BlockSpec((blk, 16), lambda i: (i, 0)),
        ],
        out_shape=[
            jax.ShapeDtypeStruct((n, d2 + 16), F32),
            jax.ShapeDtypeStruct((n, 16), F32),
        ],
    )(comb, xwa1, ad1, b1, E16, W2, Asm2, Adm2)


def _tc_final_body(comb_ref, xwa2_ref, ad2_ref, b2_ref, e1_ref, out_ref):
    d2 = out_ref.shape[1]
    a = xwa2_ref[:, d2:] + ad2_ref[...]
    wself = jnp.exp(jnp.where(a > 0, a, 0.2 * a))
    den16 = comb_ref[0, :, d2:] + comb_ref[1, :, d2:] + wself
    den = jnp.dot(den16, e1_ref[...], preferred_element_type=F32)
    wself_x = jnp.dot(wself, e1_ref[...], preferred_element_type=F32)
    num = (comb_ref[0, :, :d2] + comb_ref[1, :, :d2]
           + xwa2_ref[:, :d2] * wself_x)
    out_ref[...] = num / den + b2_ref[...]


def _run_tc_final(comb, xwa2, ad2, b2, E1, blk):
    n = xwa2.shape[0]
    d2 = xwa2.shape[1] - 16
    grid = n // blk
    return pl.pallas_call(
        _tc_final_body,
        grid=(grid,),
        in_specs=[
            pl.BlockSpec((2, blk, d2 + 16), lambda i: (0, i, 0)),
            pl.BlockSpec((blk, d2 + 16), lambda i: (i, 0)),
            pl.BlockSpec((blk, 16), lambda i: (i, 0)),
            pl.BlockSpec((1, d2), lambda i: (0, 0)),
            pl.BlockSpec((16, d2), lambda i: (0, 0)),
        ],
        out_specs=pl.BlockSpec((blk, d2), lambda i: (i, 0)),
        out_shape=jax.ShapeDtypeStruct((n, d2), F32),
    )(comb, xwa2, ad2, b2, E1)


def _make_edge_kernel(n, n_pad, e_total, d, nheads):
    e_per_w = e_total // _NW
    K = 80
    n_chunks = e_per_w // K
    rows_per_tile = n_pad // _NS
    nblk = d // 16
    dc = d + 16
    mesh = plsc.VectorSubcoreMesh(core_axis_name="c", subcore_axis_name="s")

    @functools.partial(
        pl.kernel,
        mesh=mesh,
        compiler_params=pltpu.CompilerParams(
            needs_layout_passes=False, use_tc_tiling_on_sc=False),
        out_type=jax.ShapeDtypeStruct((_NC, n_pad, dc), F32),
        scratch_types=[
            pltpu.VMEM((2, K), jnp.int32),
            pltpu.VMEM((n_chunks, K), jnp.int32),
            pltpu.VMEM((2, K, 16), F32),
            pltpu.VMEM((2, K, dc), F32),
            pltpu.VMEM_SHARED((n_pad, dc), F32),
            pltpu.SemaphoreType.DMA,
            pltpu.SemaphoreType.DMA,
            pltpu.SemaphoreType.DMA,
            pltpu.SemaphoreType.DMA,
            pltpu.SemaphoreType.DMA,
            pltpu.SemaphoreType.DMA,
        ],
    )
    def edge_kernel(src_hbm, dst_hbm, xwa_hbm, ad_hbm, zcomb_hbm, comb_out,
                    srcB, dstA, adb, xwb, comb_acc,
                    semg0, semg1, semi0, semi1, sems0, sems1):
        cid = lax.axis_index("c")
        sid = lax.axis_index("s")
        wid = sid * _NC + cid
        r0 = sid * rows_per_tile
        pltpu.sync_copy(zcomb_hbm.at[pl.ds(r0, rows_per_tile)],
                        comb_acc.at[pl.ds(r0, rows_per_tile)])
        pltpu.sync_copy(dst_hbm.at[wid], dstA)
        plsc.subcore_barrier()

        semg = (semg0, semg1)
        semi = (semi0, semi1)
        sems = (sems0, sems1)

        def idx_issue(p, ci):
            pltpu.async_copy(src_hbm.at[wid, ci], srcB.at[p], semi[p])

        def idx_wait(p):
            pltpu.make_async_copy(src_hbm.at[wid, 0], srcB.at[p],
                                  semi[p]).wait()

        def premask(b, ci):
            for j in range(K // 16):
                s = srcB[b, pl.ds(j * 16, 16)]
                t = dstA[ci, pl.ds(j * 16, 16)]
                dstA[ci, pl.ds(j * 16, 16)] = jnp.where(s == t, n, t)

        def gath(b, ci):
            sem = semg[b]
            pltpu.async_copy(xwa_hbm.at[srcB.at[b]], xwb.at[b], sem)
            pltpu.async_copy(ad_hbm.at[dstA.at[ci]], adb.at[b], sem)

        def gwait(b):
            sem = semg[b]
            pltpu.make_async_copy(xwa_hbm.at[srcB.at[0]], xwb.at[b],
                                  sem).wait()
            pltpu.make_async_copy(ad_hbm.at[dstA.at[0]], adb.at[b],
                                  sem).wait()

        def compute(b, ci):
            def edge(ei, ecarry):
                a = xwb[b, ei, pl.ds(d, 16)] + adb[b, ei, :]
                w = jnp.exp(jnp.where(a > 0.0, a, 0.2 * a))
                xwb[b, ei, pl.ds(d, 16)] = w
                for h in range(nblk):
                    hsel = h if nheads > 1 else 0
                    cf = lax.gather(
                        w, jnp.full((16, 1), hsel, jnp.int32),
                        lax.GatherDimensionNumbers(
                            offset_dims=(), collapsed_slice_dims=(0,),
                            start_index_map=(0,)),
                        (1,), mode=lax.GatherScatterMode.PROMISE_IN_BOUNDS)
                    xwb[b, ei, pl.ds(h * 16, 16)] = (
                        xwb[b, ei, pl.ds(h * 16, 16)] * cf)
                return ecarry

            lax.fori_loop(0, K, edge, 0, unroll=8)
            pltpu.async_copy(xwb.at[b], comb_acc.at[dstA.at[ci]], sems[b],
                             add=True)

        def scwait(b):
            pltpu.make_async_copy(xwb.at[b], comb_acc.at[dstA.at[0]],
                                  sems[b]).wait()

        last = n_chunks - 1
        idx_issue(0, 0)
        idx_wait(0)
        premask(0, 0)
        gath(0, 0)
        pltpu.async_copy(zcomb_hbm.at[pl.ds(0, K)], xwb.at[1], sems1)
        idx_issue(1, 1)

        def pair(i, carry):
            idx_wait(1)
            premask(1, 2 * i + 1)
            scwait(1)
            gath(1, 2 * i + 1)
            gwait(0)
            idx_issue(0, 2 * i + 2)
            compute(0, 2 * i)
            idx_wait(0)
            premask(0, 2 * i + 2)
            scwait(0)
            gath(0, 2 * i + 2)
            gwait(1)
            idx_issue(1, jnp.minimum(2 * i + 3, last))
            compute(1, 2 * i + 1)
            return carry

        lax.fori_loop(0, (n_chunks - 1) // 2, pair, 0)
        gwait(0)
        compute(0, last)
        idx_wait(1)
        scwait(0)
        scwait(1)

        plsc.subcore_barrier()
        pltpu.sync_copy(comb_acc.at[pl.ds(r0, rows_per_tile)],
                        comb_out.at[cid, pl.ds(r0, rows_per_tile)])

    return edge_kernel


def kernel(x, edge_index, W1, a_src1, a_dst1, b1, W2, a_src2, a_dst2, b2):
    n, f_in = x.shape
    e_total = edge_index.shape[1]
    h1, c1 = a_src1.shape[1], a_src1.shape[2]
    d1 = h1 * c1
    d2 = a_src2.shape[2]
    n_pad = -(-(n + 1) // (_NS * 8)) * (_NS * 8)
    blk = n_pad // 8

    src = edge_index[0].astype(jnp.int32)
    dst = edge_index[1].astype(jnp.int32)
    kchunk = 80
    src3 = src.reshape(_NW, e_total // (_NW * kchunk), kchunk)
    dst3 = dst.reshape(_NW, e_total // (_NW * kchunk), kchunk)

    rows1 = jnp.arange(d1)
    cols1 = jnp.repeat(jnp.arange(h1), c1)
    Asm1 = jnp.zeros((d1, 16), F32).at[rows1, cols1].set(a_src1.reshape(-1))
    Adm1 = jnp.zeros((d1, 16), F32).at[rows1, cols1].set(a_dst1.reshape(-1))
    Asm2 = jnp.zeros((d2, 16), F32).at[:, 0].set(a_src2.reshape(-1))
    Adm2 = jnp.zeros((d2, 16), F32).at[:, 0].set(a_dst2.reshape(-1))
    E16 = jnp.zeros((16, d1), F32).at[cols1, rows1].set(1.0)
    E1 = jnp.zeros((16, d2), F32).at[0, :].set(1.0)

    x_pad = jnp.concatenate([x, jnp.zeros((n_pad - n, f_in), F32)], axis=0)
    zcomb1 = jnp.zeros((n_pad, d1 + 16), F32)
    zcomb2 = jnp.zeros((n_pad, d2 + 16), F32)

    xwa1, ad1 = _run_tc1(x_pad, W1, Asm1, Adm1, blk)
    comb1 = _make_edge_kernel(n, n_pad, e_total, d1, h1)(
        src3, dst3, xwa1, ad1, zcomb1)
    xwa2, ad2 = _run_tc_mid(comb1, xwa1, ad1, b1.reshape(1, -1), E16,
                            W2, Asm2, Adm2, blk)
    comb2 = _make_edge_kernel(n, n_pad, e_total, d2, 1)(
        src3, dst3, xwa2, ad2, zcomb2)
    out = _run_tc_final(comb2, xwa2, ad2, b2.reshape(1, -1), E1, blk)
    return out[:n]


# speedup vs baseline: 1.3615x; 1.3615x over previous
import functools

import jax
import jax.numpy as jnp
from jax import lax
from jax.experimental import pallas as pl
from jax.experimental.pallas import tpu as pltpu
from jax.experimental.pallas import tpu_sc as plsc

F32 = jnp.float32

_NC = 2
_NS = 16
_NW = _NC * _NS


def _tc1_body(x_ref, w_ref, asm_ref, adm_ref, xwa_ref, ad_ref):
    d = w_ref.shape[1]
    xw = jnp.dot(x_ref[...], w_ref[...], preferred_element_type=F32)
    xwa_ref[:, :d] = xw
    xwa_ref[:, d:] = jnp.dot(xw, asm_ref[...], preferred_element_type=F32)
    ad_ref[...] = jnp.dot(xw, adm_ref[...], preferred_element_type=F32)


def _run_tc1(x, W, Asm, Adm, blk):
    n, f_in = x.shape
    d = W.shape[1]
    grid = n // blk
    return pl.pallas_call(
        _tc1_body,
        grid=(grid,),
        in_specs=[
            pl.BlockSpec((blk, f_in), lambda i: (i, 0)),
            pl.BlockSpec((f_in, d), lambda i: (0, 0)),
            pl.BlockSpec((d, 16), lambda i: (0, 0)),
            pl.BlockSpec((d, 16), lambda i: (0, 0)),
        ],
        out_specs=[
            pl.BlockSpec((blk, d + 16), lambda i: (i, 0)),
            pl.BlockSpec((blk, 16), lambda i: (i, 0)),
        ],
        out_shape=[
            jax.ShapeDtypeStruct((n, d + 16), F32),
            jax.ShapeDtypeStruct((n, 16), F32),
        ],
    )(x, W, Asm, Adm)


def _tc_mid_body(comb_ref, xwa1_ref, ad1_ref, b1_ref, e16_ref, w2_ref,
                 asm2_ref, adm2_ref, xwa2_ref, ad2_ref):
    d1 = w2_ref.shape[0]
    d2 = w2_ref.shape[1]
    a = xwa1_ref[:, d1:] + ad1_ref[...]
    wself = jnp.exp(jnp.where(a > 0, a, 0.2 * a))
    den16 = comb_ref[0, :, d1:] + comb_ref[1, :, d1:] + wself
    wself_x = jnp.dot(wself, e16_ref[...], preferred_element_type=F32)
    num = (comb_ref[0, :, :d1] + comb_ref[1, :, :d1]
           + xwa1_ref[:, :d1] * wself_x)
    den = jnp.dot(den16, e16_ref[...], preferred_element_type=F32)
    h = num / den + b1_ref[...]
    h = jnp.where(h > 0, h, jnp.exp(h) - 1.0)
    xw2 = jnp.dot(h, w2_ref[...], preferred_element_type=F32)
    xwa2_ref[:, :d2] = xw2
    xwa2_ref[:, d2:] = jnp.dot(xw2, asm2_ref[...], preferred_element_type=F32)
    ad2_ref[...] = jnp.dot(xw2, adm2_ref[...], preferred_element_type=F32)


def _run_tc_mid(comb, xwa1, ad1, b1, E16, W2, Asm2, Adm2, blk):
    n = xwa1.shape[0]
    d1, d2 = W2.shape
    grid = n // blk
    return pl.pallas_call(
        _tc_mid_body,
        grid=(grid,),
        in_specs=[
            pl.BlockSpec((2, blk, d1 + 16), lambda i: (0, i, 0)),
            pl.BlockSpec((blk, d1 + 16), lambda i: (i, 0)),
            pl.BlockSpec((blk, 16), lambda i: (i, 0)),
            pl.BlockSpec((1, d1), lambda i: (0, 0)),
            pl.BlockSpec((16, d1), lambda i: (0, 0)),
            pl.BlockSpec((d1, d2), lambda i: (0, 0)),
            pl.BlockSpec((d2, 16), lambda i: (0, 0)),
            pl.BlockSpec((d2, 16), lambda i: (0, 0)),
        ],
        out_specs=[
            pl.BlockSpec((blk, d2 + 16), lambda i: (i, 0)),
            pl.BlockSpec((blk, 16), lambda i: (i, 0)),
        ],
        out_shape=[
            jax.ShapeDtypeStruct((n, d2 + 16), F32),
            jax.ShapeDtypeStruct((n, 16), F32),
        ],
    )(comb, xwa1, ad1, b1, E16, W2, Asm2, Adm2)


def _tc_final_body(comb_ref, xwa2_ref, ad2_ref, b2_ref, e1_ref, out_ref):
    d2 = out_ref.shape[1]
    a = xwa2_ref[:, d2:] + ad2_ref[...]
    wself = jnp.exp(jnp.where(a > 0, a, 0.2 * a))
    den16 = comb_ref[0, :, d2:] + comb_ref[1, :, d2:] + wself
    den = jnp.dot(den16, e1_ref[...], preferred_element_type=F32)
    wself_x = jnp.dot(wself, e1_ref[...], preferred_element_type=F32)
    num = (comb_ref[0, :, :d2] + comb_ref[1, :, :d2]
           + xwa2_ref[:, :d2] * wself_x)
    out_ref[...] = num / den + b2_ref[...]


def _run_tc_final(comb, xwa2, ad2, b2, E1, blk):
    n = xwa2.shape[0]
    d2 = xwa2.shape[1] - 16
    grid = n // blk
    return pl.pallas_call(
        _tc_final_body,
        grid=(grid,),
        in_specs=[
            pl.BlockSpec((2, blk, d2 + 16), lambda i: (0, i, 0)),
            pl.BlockSpec((blk, d2 + 16), lambda i: (i, 0)),
            pl.BlockSpec((blk, 16), lambda i: (i, 0)),
            pl.BlockSpec((1, d2), lambda i: (0, 0)),
            pl.BlockSpec((16, d2), lambda i: (0, 0)),
        ],
        out_specs=pl.BlockSpec((blk, d2), lambda i: (i, 0)),
        out_shape=jax.ShapeDtypeStruct((n, d2), F32),
    )(comb, xwa2, ad2, b2, E1)


def _make_edge_kernel(n, n_pad, e_total, d, nheads):
    e_per_w = e_total // _NW
    K = 80
    n_chunks = e_per_w // K
    rows_per_tile = n_pad // _NS
    nblk = d // 16
    dc = d + 16
    mesh = plsc.VectorSubcoreMesh(core_axis_name="c", subcore_axis_name="s")

    @functools.partial(
        pl.kernel,
        mesh=mesh,
        compiler_params=pltpu.CompilerParams(
            needs_layout_passes=False, use_tc_tiling_on_sc=False),
        out_type=jax.ShapeDtypeStruct((_NC, n_pad, dc), F32),
        scratch_types=[
            pltpu.VMEM((2, K), jnp.int32),
            pltpu.VMEM((n_chunks, K), jnp.int32),
            pltpu.VMEM((2, K, 16), F32),
            pltpu.VMEM((2, K, dc), F32),
            pltpu.VMEM_SHARED((n_pad, dc), F32),
            pltpu.SemaphoreType.DMA,
            pltpu.SemaphoreType.DMA,
            pltpu.SemaphoreType.DMA,
            pltpu.SemaphoreType.DMA,
            pltpu.SemaphoreType.DMA,
            pltpu.SemaphoreType.DMA,
        ],
    )
    def edge_kernel(src_hbm, dst_hbm, xwa_hbm, ad_hbm, zcomb_hbm, comb_out,
                    srcB, dstA, adb, xwb, comb_acc,
                    semg0, semg1, semi0, semi1, sems0, sems1):
        cid = lax.axis_index("c")
        sid = lax.axis_index("s")
        wid = sid * _NC + cid
        r0 = sid * rows_per_tile
        pltpu.sync_copy(zcomb_hbm.at[pl.ds(r0, rows_per_tile)],
                        comb_acc.at[pl.ds(r0, rows_per_tile)])
        pltpu.sync_copy(dst_hbm.at[wid], dstA)
        plsc.subcore_barrier()

        semg = (semg0, semg1)
        semi = (semi0, semi1)
        sems = (sems0, sems1)

        def idx_issue(p, ci):
            pltpu.async_copy(src_hbm.at[wid, ci], srcB.at[p], semi[p])

        def idx_wait(p):
            pltpu.make_async_copy(src_hbm.at[wid, 0], srcB.at[p],
                                  semi[p]).wait()

        def premask(b, ci):
            for j in range(K // 16):
                s = srcB[b, pl.ds(j * 16, 16)]
                t = dstA[ci, pl.ds(j * 16, 16)]
                dstA[ci, pl.ds(j * 16, 16)] = jnp.where(s == t, n, t)

        def gath(b, ci):
            sem = semg[b]
            pltpu.async_copy(xwa_hbm.at[srcB.at[b]], xwb.at[b], sem)
            pltpu.async_copy(ad_hbm.at[dstA.at[ci]], adb.at[b], sem)

        def gwait(b):
            sem = semg[b]
            pltpu.make_async_copy(xwa_hbm.at[srcB.at[0]], xwb.at[b],
                                  sem).wait()
            pltpu.make_async_copy(ad_hbm.at[dstA.at[0]], adb.at[b],
                                  sem).wait()

        def compute(b, ci):
            def edge(ei, ecarry):
                a = xwb[b, ei, pl.ds(d, 16)] + adb[b, ei, :]
                w = jnp.exp(jnp.where(a > 0.0, a, 0.2 * a))
                xwb[b, ei, pl.ds(d, 16)] = w
                for h in range(nblk):
                    hsel = h if nheads > 1 else 0
                    cf = lax.gather(
                        w, jnp.full((16, 1), hsel, jnp.int32),
                        lax.GatherDimensionNumbers(
                            offset_dims=(), collapsed_slice_dims=(0,),
                            start_index_map=(0,)),
                        (1,), mode=lax.GatherScatterMode.PROMISE_IN_BOUNDS)
                    xwb[b, ei, pl.ds(h * 16, 16)] = (
                        xwb[b, ei, pl.ds(h * 16, 16)] * cf)
                return ecarry

            lax.fori_loop(0, K, edge, 0, unroll=2)
            pltpu.async_copy(xwb.at[b], comb_acc.at[dstA.at[ci]], sems[b],
                             add=True)

        def scwait(b):
            pltpu.make_async_copy(xwb.at[b], comb_acc.at[dstA.at[0]],
                                  sems[b]).wait()

        last = n_chunks - 1
        idx_issue(0, 0)
        idx_wait(0)
        premask(0, 0)
        gath(0, 0)
        pltpu.async_copy(zcomb_hbm.at[pl.ds(0, K)], xwb.at[1], sems1)
        idx_issue(1, 1)

        def pair(i, carry):
            idx_wait(1)
            premask(1, 2 * i + 1)
            scwait(1)
            gath(1, 2 * i + 1)
            gwait(0)
            idx_issue(0, 2 * i + 2)
            compute(0, 2 * i)
            idx_wait(0)
            premask(0, 2 * i + 2)
            scwait(0)
            gath(0, 2 * i + 2)
            gwait(1)
            idx_issue(1, jnp.minimum(2 * i + 3, last))
            compute(1, 2 * i + 1)
            return carry

        lax.fori_loop(0, (n_chunks - 1) // 2, pair, 0)
        gwait(0)
        compute(0, last)
        idx_wait(1)
        scwait(0)
        scwait(1)

        plsc.subcore_barrier()
        pltpu.sync_copy(comb_acc.at[pl.ds(r0, rows_per_tile)],
                        comb_out.at[cid, pl.ds(r0, rows_per_tile)])

    return edge_kernel


def kernel(x, edge_index, W1, a_src1, a_dst1, b1, W2, a_src2, a_dst2, b2):
    n, f_in = x.shape
    e_total = edge_index.shape[1]
    h1, c1 = a_src1.shape[1], a_src1.shape[2]
    d1 = h1 * c1
    d2 = a_src2.shape[2]
    n_pad = -(-(n + 1) // (_NS * 8)) * (_NS * 8)
    blk = n_pad // 8

    src = edge_index[0].astype(jnp.int32)
    dst = edge_index[1].astype(jnp.int32)
    kchunk = 80
    src3 = src.reshape(_NW, e_total // (_NW * kchunk), kchunk)
    dst3 = dst.reshape(_NW, e_total // (_NW * kchunk), kchunk)

    rows1 = jnp.arange(d1)
    cols1 = jnp.repeat(jnp.arange(h1), c1)
    Asm1 = jnp.zeros((d1, 16), F32).at[rows1, cols1].set(a_src1.reshape(-1))
    Adm1 = jnp.zeros((d1, 16), F32).at[rows1, cols1].set(a_dst1.reshape(-1))
    Asm2 = jnp.zeros((d2, 16), F32).at[:, 0].set(a_src2.reshape(-1))
    Adm2 = jnp.zeros((d2, 16), F32).at[:, 0].set(a_dst2.reshape(-1))
    E16 = jnp.zeros((16, d1), F32).at[cols1, rows1].set(1.0)
    E1 = jnp.zeros((16, d2), F32).at[0, :].set(1.0)

    x_pad = jnp.concatenate([x, jnp.zeros((n_pad - n, f_in), F32)], axis=0)
    zcomb1 = jnp.zeros((n_pad, d1 + 16), F32)
    zcomb2 = jnp.zeros((n_pad, d2 + 16), F32)

    xwa1, ad1 = _run_tc1(x_pad, W1, Asm1, Adm1, blk)
    comb1 = _make_edge_kernel(n, n_pad, e_total, d1, h1)(
        src3, dst3, xwa1, ad1, zcomb1)
    xwa2, ad2 = _run_tc_mid(comb1, xwa1, ad1, b1.reshape(1, -1), E16,
                            W2, Asm2, Adm2, blk)
    comb2 = _make_edge_kernel(n, n_pad, e_total, d2, 1)(
        src3, dst3, xwa2, ad2, zcomb2)
    out = _run_tc_final(comb2, xwa2, ad2, b2.reshape(1, -1), E1, blk)
    return out[:n]


# speedup vs baseline: 2.0444x; 1.5016x over previous
import functools

import jax
import jax.numpy as jnp
from jax import lax
from jax.experimental import pallas as pl
from jax.experimental.pallas import tpu as pltpu
from jax.experimental.pallas import tpu_sc as plsc

F32 = jnp.float32

_NC = 2
_NS = 16
_NW = _NC * _NS


def _tc1_body(x_ref, w_ref, asm_ref, adm_ref, xwa_ref, ad_ref):
    d = w_ref.shape[1]
    xw = jnp.dot(x_ref[...], w_ref[...], preferred_element_type=F32)
    xwa_ref[:, :d] = xw
    xwa_ref[:, d:] = jnp.dot(xw, asm_ref[...], preferred_element_type=F32)
    ad_ref[...] = jnp.dot(xw, adm_ref[...], preferred_element_type=F32)


def _run_tc1(x, W, Asm, Adm, blk):
    n, f_in = x.shape
    d = W.shape[1]
    grid = n // blk
    return pl.pallas_call(
        _tc1_body,
        grid=(grid,),
        in_specs=[
            pl.BlockSpec((blk, f_in), lambda i: (i, 0)),
            pl.BlockSpec((f_in, d), lambda i: (0, 0)),
            pl.BlockSpec((d, 16), lambda i: (0, 0)),
            pl.BlockSpec((d, 16), lambda i: (0, 0)),
        ],
        out_specs=[
            pl.BlockSpec((blk, d + 16), lambda i: (i, 0)),
            pl.BlockSpec((blk, 16), lambda i: (i, 0)),
        ],
        out_shape=[
            jax.ShapeDtypeStruct((n, d + 16), F32),
            jax.ShapeDtypeStruct((n, 16), F32),
        ],
    )(x, W, Asm, Adm)


def _tc_mid_body(comb_ref, xwa1_ref, ad1_ref, b1_ref, e16_ref, w2_ref,
                 asm2_ref, adm2_ref, xwa2_ref, ad2_ref):
    d1 = w2_ref.shape[0]
    d2 = w2_ref.shape[1]
    a = xwa1_ref[:, d1:] + ad1_ref[...]
    wself = jnp.exp(jnp.where(a > 0, a, 0.2 * a))
    den16 = comb_ref[0, :, d1:] + comb_ref[1, :, d1:] + wself
    wself_x = jnp.dot(wself, e16_ref[...], preferred_element_type=F32)
    num = (comb_ref[0, :, :d1] + comb_ref[1, :, :d1]
           + xwa1_ref[:, :d1] * wself_x)
    den = jnp.dot(den16, e16_ref[...], preferred_element_type=F32)
    h = num / den + b1_ref[...]
    h = jnp.where(h > 0, h, jnp.exp(h) - 1.0)
    xw2 = jnp.dot(h, w2_ref[...], preferred_element_type=F32)
    xwa2_ref[:, :d2] = xw2
    xwa2_ref[:, d2:] = jnp.dot(xw2, asm2_ref[...], preferred_element_type=F32)
    ad2_ref[...] = jnp.dot(xw2, adm2_ref[...], preferred_element_type=F32)


def _run_tc_mid(comb, xwa1, ad1, b1, E16, W2, Asm2, Adm2, blk):
    n = xwa1.shape[0]
    d1, d2 = W2.shape
    grid = n // blk
    return pl.pallas_call(
        _tc_mid_body,
        grid=(grid,),
        in_specs=[
            pl.BlockSpec((2, blk, d1 + 16), lambda i: (0, i, 0)),
            pl.BlockSpec((blk, d1 + 16), lambda i: (i, 0)),
            pl.BlockSpec((blk, 16), lambda i: (i, 0)),
            pl.BlockSpec((1, d1), lambda i: (0, 0)),
            pl.BlockSpec((16, d1), lambda i: (0, 0)),
            pl.BlockSpec((d1, d2), lambda i: (0, 0)),
            pl.BlockSpec((d2, 16), lambda i: (0, 0)),
            pl.BlockSpec((d2, 16), lambda i: (0, 0)),
        ],
        out_specs=[
            pl.BlockSpec((blk, d2 + 16), lambda i: (i, 0)),
            pl.BlockSpec((blk, 16), lambda i: (i, 0)),
        ],
        out_shape=[
            jax.ShapeDtypeStruct((n, d2 + 16), F32),
            jax.ShapeDtypeStruct((n, 16), F32),
        ],
    )(comb, xwa1, ad1, b1, E16, W2, Asm2, Adm2)


def _tc_final_body(comb_ref, xwa2_ref, ad2_ref, b2_ref, e1_ref, out_ref):
    d2 = out_ref.shape[1]
    a = xwa2_ref[:, d2:] + ad2_ref[...]
    wself = jnp.exp(jnp.where(a > 0, a, 0.2 * a))
    den16 = comb_ref[0, :, d2:] + comb_ref[1, :, d2:] + wself
    den = jnp.dot(den16, e1_ref[...], preferred_element_type=F32)
    wself_x = jnp.dot(wself, e1_ref[...], preferred_element_type=F32)
    num = (comb_ref[0, :, :d2] + comb_ref[1, :, :d2]
           + xwa2_ref[:, :d2] * wself_x)
    out_ref[...] = num / den + b2_ref[...]


def _run_tc_final(comb, xwa2, ad2, b2, E1, blk):
    n = xwa2.shape[0]
    d2 = xwa2.shape[1] - 16
    grid = n // blk
    return pl.pallas_call(
        _tc_final_body,
        grid=(grid,),
        in_specs=[
            pl.BlockSpec((2, blk, d2 + 16), lambda i: (0, i, 0)),
            pl.BlockSpec((blk, d2 + 16), lambda i: (i, 0)),
            pl.BlockSpec((blk, 16), lambda i: (i, 0)),
            pl.BlockSpec((1, d2), lambda i: (0, 0)),
            pl.BlockSpec((16, d2), lambda i: (0, 0)),
        ],
        out_specs=pl.BlockSpec((blk, d2), lambda i: (i, 0)),
        out_shape=jax.ShapeDtypeStruct((n, d2), F32),
    )(comb, xwa2, ad2, b2, E1)


def _make_edge_kernel(n, n_pad, e_total, d, nheads):
    e_per_w = e_total // _NW
    K = 80
    n_chunks = e_per_w // K
    rows_per_tile = n_pad // _NS
    nblk = d // 16
    dc = d + 16
    mesh = plsc.VectorSubcoreMesh(core_axis_name="c", subcore_axis_name="s")

    @functools.partial(
        pl.kernel,
        mesh=mesh,
        compiler_params=pltpu.CompilerParams(
            needs_layout_passes=False, use_tc_tiling_on_sc=False),
        out_type=jax.ShapeDtypeStruct((_NC, n_pad, dc), F32),
        scratch_types=[
            pltpu.VMEM((2, K), jnp.int32),
            pltpu.VMEM((n_chunks, K), jnp.int32),
            pltpu.VMEM((2, K, 16), F32),
            pltpu.VMEM((2, K, dc), F32),
            pltpu.VMEM_SHARED((n_pad, dc), F32),
            pltpu.SemaphoreType.DMA,
            pltpu.SemaphoreType.DMA,
            pltpu.SemaphoreType.DMA,
            pltpu.SemaphoreType.DMA,
            pltpu.SemaphoreType.DMA,
            pltpu.SemaphoreType.DMA,
        ],
    )
    def edge_kernel(src_hbm, dst_hbm, xwa_hbm, ad_hbm, zcomb_hbm, comb_out,
                    srcB, dstA, adb, xwb, comb_acc,
                    semg0, semg1, semi0, semi1, sems0, sems1):
        cid = lax.axis_index("c")
        sid = lax.axis_index("s")
        wid = sid * _NC + cid
        r0 = sid * rows_per_tile
        pltpu.sync_copy(zcomb_hbm.at[pl.ds(r0, rows_per_tile)],
                        comb_acc.at[pl.ds(r0, rows_per_tile)])
        pltpu.sync_copy(dst_hbm.at[wid], dstA)
        plsc.subcore_barrier()

        semg = (semg0, semg1)
        semi = (semi0, semi1)
        sems = (sems0, sems1)

        def idx_issue(p, ci):
            pltpu.async_copy(src_hbm.at[wid, ci], srcB.at[p], semi[p])

        def idx_wait(p):
            pltpu.make_async_copy(src_hbm.at[wid, 0], srcB.at[p],
                                  semi[p]).wait()

        def premask(b, ci):
            for j in range(K // 16):
                s = srcB[b, pl.ds(j * 16, 16)]
                t = dstA[ci, pl.ds(j * 16, 16)]
                dstA[ci, pl.ds(j * 16, 16)] = jnp.where(s == t, n, t)

        def gath(b, ci):
            sem = semg[b]
            pltpu.async_copy(xwa_hbm.at[srcB.at[b]], xwb.at[b], sem)
            pltpu.async_copy(ad_hbm.at[dstA.at[ci]], adb.at[b], sem)

        def gwait(b):
            sem = semg[b]
            pltpu.make_async_copy(xwa_hbm.at[srcB.at[0]], xwb.at[b],
                                  sem).wait()
            pltpu.make_async_copy(ad_hbm.at[dstA.at[0]], adb.at[b],
                                  sem).wait()

        def compute(b, ci):
            @plsc.parallel_loop(0, K, unroll=2)
            def _edge(ei):
                a = xwb[b, ei, pl.ds(d, 16)] + adb[b, ei, :]
                w = jnp.exp(jnp.where(a > 0.0, a, 0.2 * a))
                xwb[b, ei, pl.ds(d, 16)] = w
                for h in range(nblk):
                    hsel = h if nheads > 1 else 0
                    cf = lax.gather(
                        w, jnp.full((16, 1), hsel, jnp.int32),
                        lax.GatherDimensionNumbers(
                            offset_dims=(), collapsed_slice_dims=(0,),
                            start_index_map=(0,)),
                        (1,), mode=lax.GatherScatterMode.PROMISE_IN_BOUNDS)
                    xwb[b, ei, pl.ds(h * 16, 16)] = (
                        xwb[b, ei, pl.ds(h * 16, 16)] * cf)
            pltpu.async_copy(xwb.at[b], comb_acc.at[dstA.at[ci]], sems[b],
                             add=True)

        def scwait(b):
            pltpu.make_async_copy(xwb.at[b], comb_acc.at[dstA.at[0]],
                                  sems[b]).wait()

        last = n_chunks - 1
        idx_issue(0, 0)
        idx_wait(0)
        premask(0, 0)
        gath(0, 0)
        pltpu.async_copy(zcomb_hbm.at[pl.ds(0, K)], xwb.at[1], sems1)
        idx_issue(1, 1)

        def pair(i, carry):
            idx_wait(1)
            premask(1, 2 * i + 1)
            scwait(1)
            gath(1, 2 * i + 1)
            gwait(0)
            idx_issue(0, 2 * i + 2)
            compute(0, 2 * i)
            idx_wait(0)
            premask(0, 2 * i + 2)
            scwait(0)
            gath(0, 2 * i + 2)
            gwait(1)
            idx_issue(1, jnp.minimum(2 * i + 3, last))
            compute(1, 2 * i + 1)
            return carry

        lax.fori_loop(0, (n_chunks - 1) // 2, pair, 0)
        gwait(0)
        compute(0, last)
        idx_wait(1)
        scwait(0)
        scwait(1)

        plsc.subcore_barrier()
        pltpu.sync_copy(comb_acc.at[pl.ds(r0, rows_per_tile)],
                        comb_out.at[cid, pl.ds(r0, rows_per_tile)])

    return edge_kernel


def kernel(x, edge_index, W1, a_src1, a_dst1, b1, W2, a_src2, a_dst2, b2):
    n, f_in = x.shape
    e_total = edge_index.shape[1]
    h1, c1 = a_src1.shape[1], a_src1.shape[2]
    d1 = h1 * c1
    d2 = a_src2.shape[2]
    n_pad = -(-(n + 1) // (_NS * 8)) * (_NS * 8)
    blk = n_pad // 8

    src = edge_index[0].astype(jnp.int32)
    dst = edge_index[1].astype(jnp.int32)
    kchunk = 80
    src3 = src.reshape(_NW, e_total // (_NW * kchunk), kchunk)
    dst3 = dst.reshape(_NW, e_total // (_NW * kchunk), kchunk)

    rows1 = jnp.arange(d1)
    cols1 = jnp.repeat(jnp.arange(h1), c1)
    Asm1 = jnp.zeros((d1, 16), F32).at[rows1, cols1].set(a_src1.reshape(-1))
    Adm1 = jnp.zeros((d1, 16), F32).at[rows1, cols1].set(a_dst1.reshape(-1))
    Asm2 = jnp.zeros((d2, 16), F32).at[:, 0].set(a_src2.reshape(-1))
    Adm2 = jnp.zeros((d2, 16), F32).at[:, 0].set(a_dst2.reshape(-1))
    E16 = jnp.zeros((16, d1), F32).at[cols1, rows1].set(1.0)
    E1 = jnp.zeros((16, d2), F32).at[0, :].set(1.0)

    x_pad = jnp.concatenate([x, jnp.zeros((n_pad - n, f_in), F32)], axis=0)
    zcomb1 = jnp.zeros((n_pad, d1 + 16), F32)
    zcomb2 = jnp.zeros((n_pad, d2 + 16), F32)

    xwa1, ad1 = _run_tc1(x_pad, W1, Asm1, Adm1, blk)
    comb1 = _make_edge_kernel(n, n_pad, e_total, d1, h1)(
        src3, dst3, xwa1, ad1, zcomb1)
    xwa2, ad2 = _run_tc_mid(comb1, xwa1, ad1, b1.reshape(1, -1), E16,
                            W2, Asm2, Adm2, blk)
    comb2 = _make_edge_kernel(n, n_pad, e_total, d2, 1)(
        src3, dst3, xwa2, ad2, zcomb2)
    out = _run_tc_final(comb2, xwa2, ad2, b2.reshape(1, -1), E1, blk)
    return out[:n]
